# Initial kernel scaffold; baseline (speedup 1.0000x reference)
import jax
import jax.numpy as jnp
from jax.experimental import pallas as pl


def kernel(input, output):
    raise NotImplementedError("write your pallas kernel here")


# speedup vs baseline: 2.2695x; 2.2695x over previous
import jax
import jax.numpy as jnp
from jax.experimental import pallas as pl
from jax.experimental.pallas import tpu as pltpu

_BM = 512
_BN = 512
_N = 4096


def _chamfer_body(a_ref, b_ref, out_ref, acc_ref, rmin_ref, cmin_ref):
    b_b = pl.program_id(0)
    b_i = pl.program_id(1)
    b_j = pl.program_id(2)
    nb = pl.num_programs(0)
    ni = pl.num_programs(1)
    nj = pl.num_programs(2)

    a = a_ref[0]
    b = b_ref[0]
    an = jnp.sum(a * a, axis=1, keepdims=True)
    bn = jnp.sum(b * b, axis=1, keepdims=True)
    prod = jax.lax.dot_general(
        a, b, (((1,), (1,)), ((), ())), preferred_element_type=jnp.float32)
    d = an + bn.T - 2.0 * prod

    @pl.when((b_b == 0) & (b_i == 0) & (b_j == 0))
    def _init_acc():
        acc_ref[0, 0] = 0.0

    rpart = jnp.min(d, axis=1, keepdims=True)

    @pl.when(b_j == 0)
    def _init_rmin():
        rmin_ref[...] = rpart

    @pl.when(b_j > 0)
    def _acc_rmin():
        rmin_ref[...] = jnp.minimum(rmin_ref[...], rpart)

    @pl.when(b_j == nj - 1)
    def _fin_rmin():
        acc_ref[0, 0] += jnp.sum(rmin_ref[...])

    cpart = jnp.min(d, axis=0, keepdims=True)
    csl = (slice(None), pl.ds(b_j * _BN, _BN))

    @pl.when(b_i == 0)
    def _init_cmin():
        cmin_ref[csl] = cpart

    @pl.when(b_i > 0)
    def _acc_cmin():
        cmin_ref[csl] = jnp.minimum(cmin_ref[csl], cpart)

    @pl.when(b_i == ni - 1)
    def _fin_cmin():
        acc_ref[0, 0] += jnp.sum(cmin_ref[csl])

    @pl.when((b_b == nb - 1) & (b_i == ni - 1) & (b_j == nj - 1))
    def _write_out():
        out_ref[...] = jnp.full((1, 1), acc_ref[0, 0], jnp.float32)


def kernel(input, output):
    nb, n, k = input.shape
    res = pl.pallas_call(
        _chamfer_body,
        grid=(nb, n // _BM, n // _BN),
        in_specs=[
            pl.BlockSpec((1, _BM, k), lambda b, i, j: (b, i, 0)),
            pl.BlockSpec((1, _BN, k), lambda b, i, j: (b, j, 0)),
        ],
        out_specs=pl.BlockSpec((1, 1), lambda b, i, j: (0, 0)),
        out_shape=jax.ShapeDtypeStruct((1, 1), jnp.float32),
        scratch_shapes=[
            pltpu.SMEM((1, 1), jnp.float32),
            pltpu.VMEM((_BM, 1), jnp.float32),
            pltpu.VMEM((1, _N), jnp.float32),
        ],
    )(input, output)
    return res[0, 0]


# speedup vs baseline: 2.6004x; 1.1458x over previous
import jax
import jax.numpy as jnp
from jax.experimental import pallas as pl
from jax.experimental.pallas import tpu as pltpu

_BM = 512
_BN = 512
_BS = 128
_N = 4096
_K = 64
_KP = 128


def _prep_body(x_ref, y_ref, a2_ref, b2_ref):
    x = x_ref[0]
    y = y_ref[0]
    xn = jnp.sum(x * x, axis=1, keepdims=True)
    yn = jnp.sum(y * y, axis=1, keepdims=True)
    ones = jnp.ones((_BM, 1), jnp.float32)
    zeros = jnp.zeros((_BM, _KP - _K - 2), jnp.float32)
    a2_ref[0] = jnp.concatenate([x * -2.0, xn, ones, zeros], axis=1)
    b2_ref[0] = jnp.concatenate([y, ones, yn, zeros], axis=1)


def _main_body(a2_ref, b2_ref, out_ref, acc_ref, racc_ref, cacc_ref):
    b_b = pl.program_id(0)
    b_i = pl.program_id(1)
    b_j = pl.program_id(2)
    nb = pl.num_programs(0)
    ni = pl.num_programs(1)
    nj = pl.num_programs(2)

    @pl.when((b_b == 0) & (b_i == 0) & (b_j == 0))
    def _init_acc():
        acc_ref[0, 0] = 0.0

    @pl.when(b_j == 0)
    def _init_racc():
        racc_ref[...] = jnp.full((_BM, _BS), jnp.inf, jnp.float32)

    @pl.when((b_i == 0) & (b_j == 0))
    def _init_cacc():
        cacc_ref[...] = jnp.full((1, _N), jnp.inf, jnp.float32)

    a2 = a2_ref[0]
    for s in range(_BN // _BS):
        b2s = b2_ref[0, s * _BS:(s + 1) * _BS, :]
        d = jax.lax.dot_general(
            a2, b2s, (((1,), (1,)), ((), ())),
            preferred_element_type=jnp.float32)
        racc_ref[...] = jnp.minimum(racc_ref[...], d)
        csl = (slice(None), pl.ds(b_j * _BN + s * _BS, _BS))
        cacc_ref[csl] = jnp.minimum(cacc_ref[csl],
                                    jnp.min(d, axis=0, keepdims=True))

    @pl.when(b_j == nj - 1)
    def _fin_rows():
        acc_ref[0, 0] += jnp.sum(jnp.min(racc_ref[...], axis=1))

    @pl.when(b_i == ni - 1)
    def _fin_cols():
        acc_ref[0, 0] += jnp.sum(cacc_ref[0, pl.ds(b_j * _BN, _BN)])

    @pl.when((b_b == nb - 1) & (b_i == ni - 1) & (b_j == nj - 1))
    def _write_out():
        out_ref[...] = jnp.full((1, 1), acc_ref[0, 0], jnp.float32)


def kernel(input, output):
    nb, n, k = input.shape
    a2, b2 = pl.pallas_call(
        _prep_body,
        grid=(nb, n // _BM),
        in_specs=[
            pl.BlockSpec((1, _BM, k), lambda b, i: (b, i, 0)),
            pl.BlockSpec((1, _BM, k), lambda b, i: (b, i, 0)),
        ],
        out_specs=[
            pl.BlockSpec((1, _BM, _KP), lambda b, i: (b, i, 0)),
            pl.BlockSpec((1, _BM, _KP), lambda b, i: (b, i, 0)),
        ],
        out_shape=[
            jax.ShapeDtypeStruct((nb, n, _KP), jnp.float32),
            jax.ShapeDtypeStruct((nb, n, _KP), jnp.float32),
        ],
    )(input, output)

    res = pl.pallas_call(
        _main_body,
        grid=(nb, n // _BM, n // _BN),
        in_specs=[
            pl.BlockSpec((1, _BM, _KP), lambda b, i, j: (b, i, 0)),
            pl.BlockSpec((1, _BN, _KP), lambda b, i, j: (b, j, 0)),
        ],
        out_specs=pl.BlockSpec((1, 1), lambda b, i, j: (0, 0)),
        out_shape=jax.ShapeDtypeStruct((1, 1), jnp.float32),
        scratch_shapes=[
            pltpu.SMEM((1, 1), jnp.float32),
            pltpu.VMEM((_BM, _BS), jnp.float32),
            pltpu.VMEM((1, _N), jnp.float32),
        ],
    )(a2, b2)
    return res[0, 0]


# speedup vs baseline: 4.3049x; 1.6555x over previous
import jax
import jax.numpy as jnp
from jax.experimental import pallas as pl
from jax.experimental.pallas import tpu as pltpu

_BM = 1024
_BN = 512
_BS = 128
_N = 4096
_K = 64
_KP = 128


def _prep_body(x_ref, y_ref, a2_ref, b2_ref):
    x = x_ref[0]
    y = y_ref[0]
    xn = jnp.sum(x * x, axis=1, keepdims=True)
    yn = jnp.sum(y * y, axis=1, keepdims=True)
    ones = jnp.ones((_BM, 1), jnp.float32)
    zeros = jnp.zeros((_BM, _KP - _K - 2), jnp.float32)
    a2_ref[0] = jnp.concatenate([x * -2.0, xn, ones, zeros], axis=1)
    b2_ref[0] = jnp.concatenate([y, ones, yn, zeros], axis=1)


def _main_body(a2_ref, b2_ref, out_ref, acc_ref, racc_ref, cacc_ref):
    b_b = pl.program_id(0)
    b_i = pl.program_id(1)
    b_j = pl.program_id(2)
    nb = pl.num_programs(0)
    ni = pl.num_programs(1)
    nj = pl.num_programs(2)

    @pl.when((b_b == 0) & (b_i == 0) & (b_j == 0))
    def _init_acc():
        acc_ref[0, 0] = 0.0

    @pl.when(b_j == 0)
    def _init_racc():
        racc_ref[...] = jnp.full((_BM, _BS), jnp.inf, jnp.float32)

    @pl.when((b_i == 0) & (b_j == 0))
    def _init_cacc():
        cacc_ref[...] = jnp.full((8, _N), jnp.inf, jnp.float32)

    a2 = a2_ref[0]
    for s in range(_BN // _BS):
        b2s = b2_ref[0, s * _BS:(s + 1) * _BS, :]
        d = jax.lax.dot_general(
            a2, b2s, (((1,), (1,)), ((), ())),
            preferred_element_type=jnp.float32)
        racc_ref[...] = jnp.minimum(racc_ref[...], d)
        cp8 = jnp.min(d.reshape(_BM // 8, 8, _BS), axis=0)
        csl = (slice(None), pl.ds(b_j * _BN + s * _BS, _BS))
        cacc_ref[csl] = jnp.minimum(cacc_ref[csl], cp8)

    @pl.when(b_j == nj - 1)
    def _fin_rows():
        acc_ref[0, 0] += jnp.sum(jnp.min(racc_ref[...], axis=1))

    @pl.when(b_i == ni - 1)
    def _fin_cols():
        acc_ref[0, 0] += jnp.sum(
            jnp.min(cacc_ref[:, pl.ds(b_j * _BN, _BN)], axis=0))

    @pl.when((b_b == nb - 1) & (b_i == ni - 1) & (b_j == nj - 1))
    def _write_out():
        out_ref[...] = jnp.full((1, 1), acc_ref[0, 0], jnp.float32)


def kernel(input, output):
    nb, n, k = input.shape
    a2, b2 = pl.pallas_call(
        _prep_body,
        grid=(nb, n // _BM),
        in_specs=[
            pl.BlockSpec((1, _BM, k), lambda b, i: (b, i, 0)),
            pl.BlockSpec((1, _BM, k), lambda b, i: (b, i, 0)),
        ],
        out_specs=[
            pl.BlockSpec((1, _BM, _KP), lambda b, i: (b, i, 0)),
            pl.BlockSpec((1, _BM, _KP), lambda b, i: (b, i, 0)),
        ],
        out_shape=[
            jax.ShapeDtypeStruct((nb, n, _KP), jnp.float32),
            jax.ShapeDtypeStruct((nb, n, _KP), jnp.float32),
        ],
    )(input, output)

    res = pl.pallas_call(
        _main_body,
        grid=(nb, n // _BM, n // _BN),
        in_specs=[
            pl.BlockSpec((1, _BM, _KP), lambda b, i, j: (b, i, 0)),
            pl.BlockSpec((1, _BN, _KP), lambda b, i, j: (b, j, 0)),
        ],
        out_specs=pl.BlockSpec((1, 1), lambda b, i, j: (0, 0)),
        out_shape=jax.ShapeDtypeStruct((1, 1), jnp.float32),
        scratch_shapes=[
            pltpu.SMEM((1, 1), jnp.float32),
            pltpu.VMEM((_BM, _BS), jnp.float32),
            pltpu.VMEM((8, _N), jnp.float32),
        ],
    )(a2, b2)
    return res[0, 0]


# speedup vs baseline: 5.8868x; 1.3675x over previous
import jax
import jax.numpy as jnp
from jax.experimental import pallas as pl
from jax.experimental.pallas import tpu as pltpu

_BM = 1024
_BN = 1024
_BS = 128
_N = 4096
_K = 64
_KP = 128


def _prep_body(x_ref, y_ref, a2_ref, b2_ref):
    x = x_ref[0]
    y = y_ref[0]
    xn = jnp.sum(x * x, axis=1, keepdims=True)
    yn = jnp.sum(y * y, axis=1, keepdims=True)
    ones = jnp.ones((_BM, 1), jnp.float32)
    zeros = jnp.zeros((_BM, _KP - _K - 2), jnp.float32)
    a2_ref[0] = jnp.concatenate([x * -2.0, xn, ones, zeros], axis=1)
    b2_ref[0] = jnp.concatenate([y, ones, yn, zeros], axis=1)


def _main_body(a2_ref, b2_ref, out_ref, acc_ref, racc_ref, cacc_ref):
    b_b = pl.program_id(0)
    b_i = pl.program_id(1)
    b_j = pl.program_id(2)
    nb = pl.num_programs(0)
    ni = pl.num_programs(1)
    nj = pl.num_programs(2)

    @pl.when((b_b == 0) & (b_i == 0) & (b_j == 0))
    def _init_acc():
        acc_ref[0, 0] = 0.0

    @pl.when(b_j == 0)
    def _init_racc():
        racc_ref[...] = jnp.full((_BM, _BS), jnp.inf, jnp.float32)

    @pl.when((b_i == 0) & (b_j == 0))
    def _init_cacc():
        cacc_ref[...] = jnp.full((8, _N), jnp.inf, jnp.float32)

    a2 = a2_ref[0]
    for s in range(_BN // _BS):
        b2s = b2_ref[0, s * _BS:(s + 1) * _BS, :]
        d = jax.lax.dot_general(
            a2, b2s, (((1,), (1,)), ((), ())),
            preferred_element_type=jnp.float32)
        racc_ref[...] = jnp.minimum(racc_ref[...], d)
        cp8 = jnp.min(d.reshape(_BM // 8, 8, _BS), axis=0)
        csl = (slice(None), pl.ds(b_j * _BN + s * _BS, _BS))
        cacc_ref[csl] = jnp.minimum(cacc_ref[csl], cp8)

    @pl.when(b_j == nj - 1)
    def _fin_rows():
        acc_ref[0, 0] += jnp.sum(jnp.min(racc_ref[...], axis=1))

    @pl.when(b_i == ni - 1)
    def _fin_cols():
        acc_ref[0, 0] += jnp.sum(
            jnp.min(cacc_ref[:, pl.ds(b_j * _BN, _BN)], axis=0))

    @pl.when((b_b == nb - 1) & (b_i == ni - 1) & (b_j == nj - 1))
    def _write_out():
        out_ref[...] = jnp.full((1, 1), acc_ref[0, 0], jnp.float32)


def kernel(input, output):
    nb, n, k = input.shape
    a2, b2 = pl.pallas_call(
        _prep_body,
        grid=(nb, n // _BM),
        in_specs=[
            pl.BlockSpec((1, _BM, k), lambda b, i: (b, i, 0)),
            pl.BlockSpec((1, _BM, k), lambda b, i: (b, i, 0)),
        ],
        out_specs=[
            pl.BlockSpec((1, _BM, _KP), lambda b, i: (b, i, 0)),
            pl.BlockSpec((1, _BM, _KP), lambda b, i: (b, i, 0)),
        ],
        out_shape=[
            jax.ShapeDtypeStruct((nb, n, _KP), jnp.float32),
            jax.ShapeDtypeStruct((nb, n, _KP), jnp.float32),
        ],
    )(input, output)

    res = pl.pallas_call(
        _main_body,
        grid=(nb, n // _BM, n // _BN),
        in_specs=[
            pl.BlockSpec((1, _BM, _KP), lambda b, i, j: (b, i, 0)),
            pl.BlockSpec((1, _BN, _KP), lambda b, i, j: (b, j, 0)),
        ],
        out_specs=pl.BlockSpec((1, 1), lambda b, i, j: (0, 0)),
        out_shape=jax.ShapeDtypeStruct((1, 1), jnp.float32),
        scratch_shapes=[
            pltpu.SMEM((1, 1), jnp.float32),
            pltpu.VMEM((_BM, _BS), jnp.float32),
            pltpu.VMEM((8, _N), jnp.float32),
        ],
    )(a2, b2)
    return res[0, 0]


# speedup vs baseline: 6.1915x; 1.0518x over previous
import jax
import jax.numpy as jnp
from jax.experimental import pallas as pl
from jax.experimental.pallas import tpu as pltpu

_BM = 1024
_BN = 1024
_BS = 128
_N = 4096
_K = 64
_KP = 128


def _prep_body(x_ref, y_ref, a2_ref, b2_ref):
    x = x_ref[0]
    y = y_ref[0]
    xn = jnp.sum(x * x, axis=1, keepdims=True)
    yn = jnp.sum(y * y, axis=1, keepdims=True)
    ones = jnp.ones((_BM, 1), jnp.float32)
    zeros = jnp.zeros((_BM, _KP - _K - 2), jnp.float32)
    a2_ref[0] = jnp.concatenate(
        [x * -2.0, xn, ones, zeros], axis=1).astype(jnp.bfloat16)
    b2_ref[0] = jnp.concatenate(
        [y, ones, yn, zeros], axis=1).astype(jnp.bfloat16)


def _main_body(a2_ref, b2_ref, out_ref, acc_ref, racc_ref, cacc_ref):
    b_b = pl.program_id(0)
    b_i = pl.program_id(1)
    b_j = pl.program_id(2)
    nb = pl.num_programs(0)
    ni = pl.num_programs(1)
    nj = pl.num_programs(2)

    @pl.when((b_b == 0) & (b_i == 0) & (b_j == 0))
    def _init_acc():
        acc_ref[0, 0] = 0.0

    @pl.when(b_j == 0)
    def _init_racc():
        racc_ref[...] = jnp.full((_BM, _BS), jnp.inf, jnp.float32)

    @pl.when((b_i == 0) & (b_j == 0))
    def _init_cacc():
        cacc_ref[...] = jnp.full((8, _N), jnp.inf, jnp.float32)

    a2 = a2_ref[0]
    for s in range(_BN // _BS):
        b2s = b2_ref[0, s * _BS:(s + 1) * _BS, :]
        d = jax.lax.dot_general(
            a2, b2s, (((1,), (1,)), ((), ())),
            preferred_element_type=jnp.float32)
        racc_ref[...] = jnp.minimum(racc_ref[...], d)
        cp8 = jnp.min(d.reshape(_BM // 8, 8, _BS), axis=0)
        csl = (slice(None), pl.ds(b_j * _BN + s * _BS, _BS))
        cacc_ref[csl] = jnp.minimum(cacc_ref[csl], cp8)

    @pl.when(b_j == nj - 1)
    def _fin_rows():
        acc_ref[0, 0] += jnp.sum(jnp.min(racc_ref[...], axis=1))

    @pl.when(b_i == ni - 1)
    def _fin_cols():
        acc_ref[0, 0] += jnp.sum(
            jnp.min(cacc_ref[:, pl.ds(b_j * _BN, _BN)], axis=0))

    @pl.when((b_b == nb - 1) & (b_i == ni - 1) & (b_j == nj - 1))
    def _write_out():
        out_ref[...] = jnp.full((1, 1), acc_ref[0, 0], jnp.float32)


def kernel(input, output):
    nb, n, k = input.shape
    a2, b2 = pl.pallas_call(
        _prep_body,
        grid=(nb, n // _BM),
        in_specs=[
            pl.BlockSpec((1, _BM, k), lambda b, i: (b, i, 0)),
            pl.BlockSpec((1, _BM, k), lambda b, i: (b, i, 0)),
        ],
        out_specs=[
            pl.BlockSpec((1, _BM, _KP), lambda b, i: (b, i, 0)),
            pl.BlockSpec((1, _BM, _KP), lambda b, i: (b, i, 0)),
        ],
        out_shape=[
            jax.ShapeDtypeStruct((nb, n, _KP), jnp.bfloat16),
            jax.ShapeDtypeStruct((nb, n, _KP), jnp.bfloat16),
        ],
    )(input, output)

    res = pl.pallas_call(
        _main_body,
        grid=(nb, n // _BM, n // _BN),
        in_specs=[
            pl.BlockSpec((1, _BM, _KP), lambda b, i, j: (b, i, 0)),
            pl.BlockSpec((1, _BN, _KP), lambda b, i, j: (b, j, 0)),
        ],
        out_specs=pl.BlockSpec((1, 1), lambda b, i, j: (0, 0)),
        out_shape=jax.ShapeDtypeStruct((1, 1), jnp.float32),
        scratch_shapes=[
            pltpu.SMEM((1, 1), jnp.float32),
            pltpu.VMEM((_BM, _BS), jnp.float32),
            pltpu.VMEM((8, _N), jnp.float32),
        ],
    )(a2, b2)
    return res[0, 0]
